# baseline (speedup 1.0000x reference)
import dataclasses

import jax
import jax.numpy as jnp
from jax import lax
from jax.experimental import pallas as pl
from jax.experimental.pallas import tpu as pltpu
from jax.experimental.pallas import tpu_sc as plsc

N = 10000
D = 128
E = 320000
NC = 2
NS = 16
NW = NC * NS
EPT = E // NW
W = 64
NPAD = 10240
RPS = NPAD // NS

_PREC = jax.lax.Precision.HIGHEST


def _pre_body(left_ref, right_ref, wlt_ref, bl_ref, wrt_ref, wev_ref, sp_ref,
              l_out, r_out, wev_out):
    s = sp_ref[0, 0]
    l_out[...] = (jnp.dot(left_ref[...], wlt_ref[...],
                          preferred_element_type=jnp.float32,
                          precision=_PREC) + bl_ref[...]) * s
    r_out[...] = jnp.dot(right_ref[...], wrt_ref[...],
                         preferred_element_type=jnp.float32,
                         precision=_PREC) * s
    wev_out[...] = wev_ref[...] * s


_QB = 2000


def _pre_call(left, right, wlt, bl, wrt, wev, sp):
    full = lambda shape: pl.BlockSpec(shape, lambda i: (0,) * len(shape))
    return pl.pallas_call(
        _pre_body,
        grid=(N // _QB,),
        in_specs=[
            pl.BlockSpec((_QB, D), lambda i: (i, 0)),
            pl.BlockSpec((_QB, D), lambda i: (i, 0)),
            full((D, D)),
            full((1, D)),
            full((D, D)),
            full((1, D)),
            full((1, 1)),
        ],
        out_specs=(
            pl.BlockSpec((_QB, D), lambda i: (i, 0)),
            pl.BlockSpec((_QB, D), lambda i: (i, 0)),
            full((1, D)),
        ),
        out_shape=(
            jax.ShapeDtypeStruct((N, D), jnp.float32),
            jax.ShapeDtypeStruct((N, D), jnp.float32),
            jax.ShapeDtypeStruct((1, D), jnp.float32),
        ),
    )(left, right, wlt, bl, wrt, wev, sp)


NWINF = 156
TAIL = EPT - NWINF * W
NIT = NWINF // 2


def _sc_body(l_hbm, r_hbm, wev_hbm, si_hbm, di_hbm, ef_hbm, z_hbm,
             out_hbm, deg_hbm,
             si2_0, da_0, db_0, ef2_0,
             si2_1, da_1, db_1, ef2_1,
             l_v0, r_v0, l_v1, r_v1,
             st_t, dt_t, et_t, wev_v,
             deg_v, acc, sgi0, sgi1, sg0, sg1):
    c = lax.axis_index("c")
    s = lax.axis_index("s")
    wid = c * NS + s
    tile_base = wid * EPT

    pltpu.sync_copy(z_hbm, acc.at[pl.ds(s * RPS, RPS)])
    pltpu.sync_copy(wev_hbm, wev_v)

    zero16 = jnp.zeros((16,), jnp.float32)

    @pl.loop(0, NPAD, step=16)
    def _zero_deg(i):
        deg_v[pl.ds(i, 16)] = zero16

    plsc.subcore_barrier()

    one16 = jnp.ones((16,), jnp.float32)
    sets = ((si2_0, da_0, db_0, ef2_0, sgi0),
            (si2_1, da_1, db_1, ef2_1, sgi1))
    gbufs = ((l_v0, r_v0, sg0), (l_v1, r_v1, sg1))

    def set_issue(st, w0):
        si2, da, db, ef2, sem = st
        base = tile_base + w0 * W
        pltpu.async_copy(si_hbm.at[pl.ds(base, 2 * W)], si2, sem)
        pltpu.async_copy(di_hbm.at[pl.ds(base, W)], da, sem)
        pltpu.async_copy(di_hbm.at[pl.ds(base + W, W)], db, sem)
        pltpu.async_copy(ef_hbm.at[pl.ds(base, 2 * W)],
                         ef2.at[pl.ds(0, 2 * W)], sem)

    def set_wait(st):
        si2, da, db, ef2, sem = st
        base = tile_base
        pltpu.make_async_copy(si_hbm.at[pl.ds(base, 2 * W)], si2, sem).wait()
        pltpu.make_async_copy(di_hbm.at[pl.ds(base, W)], da, sem).wait()
        pltpu.make_async_copy(di_hbm.at[pl.ds(base, W)], db, sem).wait()
        pltpu.make_async_copy(ef_hbm.at[pl.ds(base, 2 * W)],
                              ef2.at[pl.ds(0, 2 * W)], sem).wait()

    def g_issue(st, half, gb):
        si2 = st[0]
        l_v, r_v, sem = gb
        didx = st[1] if half == 0 else st[2]
        pltpu.async_copy(l_hbm.at[si2.at[pl.ds(half * W, W)]], l_v, sem)
        pltpu.async_copy(r_hbm.at[didx], r_v, sem)

    def g_wait(st, half, gb):
        si2 = st[0]
        l_v, r_v, sem = gb
        didx = st[1] if half == 0 else st[2]
        pltpu.make_async_copy(l_hbm.at[si2.at[pl.ds(half * W, W)]],
                              l_v, sem).wait()
        pltpu.make_async_copy(r_hbm.at[didx], r_v, sem).wait()

    def compute_scatter(st, half, gb):
        _, da, db, ef2, _ = st
        l_v, r_v, _ = gb
        didx = da if half == 0 else db
        eoff = half * W

        @pl.loop(0, W, step=16)
        def _hist(i0):
            dv = didx[pl.ds(i0, 16)]
            plsc.addupdate_scatter(deg_v, [dv], one16)

        wchunks = tuple(wev_v[pl.ds(jj * 16, 16)] for jj in range(D // 16))

        @plsc.parallel_loop(0, W, unroll=8, carry=wchunks)
        def _edge(i, wc):
            e = ef2[pl.ds(eoff + i, 16)][0]
            for jj in range(D // 16):
                sl = pl.ds(jj * 16, 16)
                l_v[i, sl] = jnp.maximum(
                    l_v[i, sl] + r_v[i, sl] + e * wc[jj], 0.0)
            return wc

        pltpu.sync_copy(l_v, acc.at[didx], add=True)

    def pipe_iter(i, p, q, do_refill, do_next):
        g_issue(sets[p], 1, gbufs[1])
        g_wait(sets[p], 0, gbufs[0])
        compute_scatter(sets[p], 0, gbufs[0])
        g_wait(sets[p], 1, gbufs[1])
        compute_scatter(sets[p], 1, gbufs[1])
        if do_refill:
            set_issue(sets[p], 2 * i + 4)
        if do_next:
            set_wait(sets[q])
            g_issue(sets[q], 0, gbufs[0])

    set_issue(sets[0], 0)
    set_wait(sets[0])
    g_issue(sets[0], 0, gbufs[0])
    set_issue(sets[1], 2)

    @pl.loop(0, NIT - 2, step=2)
    def _pipe(t):
        pipe_iter(t, 0, 1, True, True)
        pipe_iter(t + 1, 1, 0, True, True)

    pipe_iter(NIT - 2, 0, 1, False, True)
    pipe_iter(NIT - 1, 1, 0, False, False)

    tb = tile_base + NWINF * W
    ct1 = pltpu.async_copy(si_hbm.at[pl.ds(tb, TAIL)], st_t, sgi0)
    ct2 = pltpu.async_copy(di_hbm.at[pl.ds(tb, TAIL)], dt_t, sgi0)
    ct3 = pltpu.async_copy(ef_hbm.at[pl.ds(tb, TAIL)],
                           et_t.at[pl.ds(0, TAIL)], sgi0)
    ct1.wait()
    ct2.wait()
    ct3.wait()
    cl = pltpu.async_copy(l_hbm.at[st_t], l_v0.at[pl.ds(0, TAIL)], sg0)
    cr = pltpu.async_copy(r_hbm.at[dt_t], r_v0.at[pl.ds(0, TAIL)], sg0)
    cl.wait()
    cr.wait()
    dvt = dt_t[pl.ds(0, 16)]
    plsc.addupdate_scatter(deg_v, [dvt], one16)

    @pl.loop(0, TAIL)
    def _tedge(i):
        e = et_t[pl.ds(i, 16)][0]
        for jj in range(D // 16):
            sl = pl.ds(jj * 16, 16)
            l_v0[i, sl] = jnp.maximum(
                l_v0[i, sl] + r_v0[i, sl] + e * wev_v[sl], 0.0)

    pltpu.sync_copy(l_v0.at[pl.ds(0, TAIL)], acc.at[dt_t], add=True)

    plsc.subcore_barrier()
    pltpu.sync_copy(acc.at[pl.ds(s * RPS, RPS)],
                    out_hbm.at[c, pl.ds(s * RPS, RPS)])
    pltpu.sync_copy(deg_v, deg_hbm.at[pl.ds(wid * NPAD, NPAD)])


def _sc_call(l_arr, r_arr, wev, si, di, ef, z):
    mesh = plsc.VectorSubcoreMesh(core_axis_name="c", subcore_axis_name="s")
    cp = pltpu.CompilerParams()
    if "needs_layout_passes" in pltpu.CompilerParams.__dataclass_fields__:
        cp = dataclasses.replace(cp, needs_layout_passes=False)
    run = pl.kernel(
        _sc_body,
        out_type=(
            jax.ShapeDtypeStruct((NC, NPAD, D), jnp.float32),
            jax.ShapeDtypeStruct((NW * NPAD,), jnp.float32),
        ),
        mesh=mesh,
        scratch_types=[
            pltpu.VMEM((2 * W,), jnp.int32),
            pltpu.VMEM((W,), jnp.int32),
            pltpu.VMEM((W,), jnp.int32),
            pltpu.VMEM((2 * W + 16,), jnp.float32),
            pltpu.VMEM((2 * W,), jnp.int32),
            pltpu.VMEM((W,), jnp.int32),
            pltpu.VMEM((W,), jnp.int32),
            pltpu.VMEM((2 * W + 16,), jnp.float32),
            pltpu.VMEM((W, D), jnp.float32),
            pltpu.VMEM((W, D), jnp.float32),
            pltpu.VMEM((W, D), jnp.float32),
            pltpu.VMEM((W, D), jnp.float32),
            pltpu.VMEM((TAIL,), jnp.int32),
            pltpu.VMEM((TAIL,), jnp.int32),
            pltpu.VMEM((TAIL + 16,), jnp.float32),
            pltpu.VMEM((D,), jnp.float32),
            pltpu.VMEM((NPAD,), jnp.float32),
            pltpu.VMEM_SHARED((NPAD, D), jnp.float32),
            pltpu.SemaphoreType.DMA,
            pltpu.SemaphoreType.DMA,
            pltpu.SemaphoreType.DMA,
            pltpu.SemaphoreType.DMA,
        ],
        compiler_params=cp,
    )
    return run(l_arr, r_arr, wev, si, di, ef, z)


def _post_body(acc_ref, degs_ref, right_ref, wft_ref, bf_ref, spost_ref,
               w1a_ref, w1b_ref, b1_ref, w2t_ref, b2_ref, out_ref):
    agg = acc_ref[0] + acc_ref[1]
    conv = jnp.dot(agg, wft_ref[...], preferred_element_type=jnp.float32,
                   precision=_PREC)
    b32 = jnp.broadcast_to(bf_ref[...], (NW, D))
    degb = lax.dot_general(degs_ref[...], b32, (((0,), (0,)), ((), ())),
                           preferred_element_type=jnp.float32,
                           precision=_PREC)
    conv = (conv + degb) * spost_ref[0, 0]
    h = jnp.dot(conv, w1a_ref[...], preferred_element_type=jnp.float32,
                precision=_PREC)
    h = h + jnp.dot(right_ref[...], w1b_ref[...],
                    preferred_element_type=jnp.float32, precision=_PREC)
    h = jnp.maximum(h + b1_ref[...], 0.0)
    out_ref[...] = jnp.dot(h, w2t_ref[...], preferred_element_type=jnp.float32,
                           precision=_PREC) + b2_ref[...]


_PB = 1024


def _post_call(acc, degs, right, wft, bf, spost, w1a, w1b, b1, w2t, b2):
    full = lambda shape: pl.BlockSpec(shape, lambda i: (0,) * len(shape))
    return pl.pallas_call(
        _post_body,
        grid=(NPAD // _PB,),
        in_specs=[
            pl.BlockSpec((NC, _PB, D), lambda i: (0, i, 0)),
            pl.BlockSpec((NW, _PB), lambda i: (0, i)),
            pl.BlockSpec((_PB, D), lambda i: (i, 0)),
            full((D, D)),
            full((1, D)),
            full((1, 1)),
            full((D, D)),
            full((D, D)),
            full((1, D)),
            full((D, D)),
            full((1, D)),
        ],
        out_specs=pl.BlockSpec((_PB, D), lambda i: (i, 0)),
        out_shape=jax.ShapeDtypeStruct((N, D), jnp.float32),
    )(acc, degs, right, wft, bf, spost, w1a, w1b, b1, w2t, b2)


def kernel(left_features, edge_indices, edge_features, right_features,
           scatter_out_size, W_left, b_left, W_edge, W_right, scale_pre,
           W_final, b_final, scale_post, W_out1, b_out1, W_out2, b_out2):
    del scatter_out_size

    l_arr, r_arr, wev = _pre_call(
        left_features, right_features,
        W_left.T, b_left.reshape(1, D), W_right.T,
        W_edge.T, scale_pre.reshape(1, 1))

    zeros = jnp.zeros((RPS, D), dtype=jnp.float32)
    acc, degs = _sc_call(l_arr, r_arr, wev.reshape(D),
                         edge_indices[0], edge_indices[1],
                         edge_features.reshape(E), zeros)

    out = _post_call(
        acc, degs.reshape(NW, NPAD), right_features,
        W_final.T, b_final.reshape(1, D), scale_post.reshape(1, 1),
        W_out1[:, :D].T, W_out1[:, D:].T, b_out1.reshape(1, D),
        W_out2.T, b_out2.reshape(1, D))
    return out


# speedup vs baseline: 1.0196x; 1.0196x over previous
import dataclasses

import jax
import jax.numpy as jnp
from jax import lax
from jax.experimental import pallas as pl
from jax.experimental.pallas import tpu as pltpu
from jax.experimental.pallas import tpu_sc as plsc

N = 10000
D = 128
E = 320000
NC = 2
NS = 16
NW = NC * NS
EPT = E // NW
W = 64
NPAD = 10240
RPS = NPAD // NS

_PREC = jax.lax.Precision.HIGHEST


def _pre_body(left_ref, right_ref, wlt_ref, bl_ref, wrt_ref, wev_ref, sp_ref,
              l_out, r_out, wev_out):
    s = sp_ref[0, 0]
    l_out[...] = (jnp.dot(left_ref[...], wlt_ref[...],
                          preferred_element_type=jnp.float32,
                          precision=_PREC) + bl_ref[...]) * s
    r_out[...] = jnp.dot(right_ref[...], wrt_ref[...],
                         preferred_element_type=jnp.float32,
                         precision=_PREC) * s
    wev_out[...] = wev_ref[...] * s


_QB = 2000


def _pre_call(left, right, wlt, bl, wrt, wev, sp):
    full = lambda shape: pl.BlockSpec(shape, lambda i: (0,) * len(shape))
    return pl.pallas_call(
        _pre_body,
        grid=(N // _QB,),
        in_specs=[
            pl.BlockSpec((_QB, D), lambda i: (i, 0)),
            pl.BlockSpec((_QB, D), lambda i: (i, 0)),
            full((D, D)),
            full((1, D)),
            full((D, D)),
            full((1, D)),
            full((1, 1)),
        ],
        out_specs=(
            pl.BlockSpec((_QB, D), lambda i: (i, 0)),
            pl.BlockSpec((_QB, D), lambda i: (i, 0)),
            full((1, D)),
        ),
        out_shape=(
            jax.ShapeDtypeStruct((N, D), jnp.float32),
            jax.ShapeDtypeStruct((N, D), jnp.float32),
            jax.ShapeDtypeStruct((1, D), jnp.float32),
        ),
    )(left, right, wlt, bl, wrt, wev, sp)


NWINF = 156
TAIL = EPT - NWINF * W
NIT = NWINF // 2


def _sc_body(l_hbm, r_hbm, wev_hbm, si_hbm, di_hbm, ef_hbm, z_hbm,
             out_hbm, deg_hbm,
             si2_0, da_0, db_0, ef2_0,
             si2_1, da_1, db_1, ef2_1,
             l_v0, r_v0, l_v1, r_v1,
             st_t, dt_t, et_t, wev_v,
             deg_v, acc, sgi0, sgi1, sg0, sg1):
    c = lax.axis_index("c")
    s = lax.axis_index("s")
    wid = c * NS + s
    tile_base = wid * EPT

    pltpu.sync_copy(z_hbm, acc.at[pl.ds(s * RPS, RPS)])
    pltpu.sync_copy(wev_hbm, wev_v)

    zero16 = jnp.zeros((16,), jnp.float32)

    @pl.loop(0, NPAD, step=16)
    def _zero_deg(i):
        deg_v[pl.ds(i, 16)] = zero16

    plsc.subcore_barrier()

    one16 = jnp.ones((16,), jnp.float32)
    sets = ((si2_0, da_0, db_0, ef2_0, sgi0),
            (si2_1, da_1, db_1, ef2_1, sgi1))
    gbufs = ((l_v0, r_v0, sg0), (l_v1, r_v1, sg1))

    def set_issue(st, w0):
        si2, da, db, ef2, sem = st
        base = tile_base + w0 * W
        pltpu.async_copy(si_hbm.at[pl.ds(base, 2 * W)], si2, sem)
        pltpu.async_copy(di_hbm.at[pl.ds(base, W)], da, sem)
        pltpu.async_copy(di_hbm.at[pl.ds(base + W, W)], db, sem)
        pltpu.async_copy(ef_hbm.at[pl.ds(base, 2 * W)],
                         ef2.at[pl.ds(0, 2 * W)], sem)

    def set_wait(st):
        si2, da, db, ef2, sem = st
        base = tile_base
        pltpu.make_async_copy(si_hbm.at[pl.ds(base, 2 * W)], si2, sem).wait()
        pltpu.make_async_copy(di_hbm.at[pl.ds(base, W)], da, sem).wait()
        pltpu.make_async_copy(di_hbm.at[pl.ds(base, W)], db, sem).wait()
        pltpu.make_async_copy(ef_hbm.at[pl.ds(base, 2 * W)],
                              ef2.at[pl.ds(0, 2 * W)], sem).wait()

    def g_issue(st, half, gb):
        si2 = st[0]
        l_v, r_v, sem = gb
        didx = st[1] if half == 0 else st[2]
        pltpu.async_copy(l_hbm.at[si2.at[pl.ds(half * W, W)]], l_v, sem)
        pltpu.async_copy(r_hbm.at[didx], r_v, sem)

    def g_wait(st, half, gb):
        si2 = st[0]
        l_v, r_v, sem = gb
        didx = st[1] if half == 0 else st[2]
        pltpu.make_async_copy(l_hbm.at[si2.at[pl.ds(half * W, W)]],
                              l_v, sem).wait()
        pltpu.make_async_copy(r_hbm.at[didx], r_v, sem).wait()

    def compute_scatter(st, half, gb):
        _, da, db, ef2, _ = st
        l_v, r_v, _ = gb
        didx = da if half == 0 else db
        eoff = half * W

        @pl.loop(0, W, step=16)
        def _hist(i0):
            dv = didx[pl.ds(i0, 16)]
            plsc.addupdate_scatter(deg_v, [dv], one16)

        wchunks = tuple(wev_v[pl.ds(jj * 16, 16)] for jj in range(D // 16))

        @plsc.parallel_loop(0, W, unroll=4, carry=wchunks)
        def _edge(i, wc):
            e = ef2[pl.ds(eoff + i, 16)][0]
            for jj in range(D // 16):
                sl = pl.ds(jj * 16, 16)
                l_v[i, sl] = jnp.maximum(
                    l_v[i, sl] + r_v[i, sl] + e * wc[jj], 0.0)
            return wc

        pltpu.sync_copy(l_v, acc.at[didx], add=True)

    def pipe_iter(i, p, q, do_refill, do_next):
        g_issue(sets[p], 1, gbufs[1])
        g_wait(sets[p], 0, gbufs[0])
        compute_scatter(sets[p], 0, gbufs[0])
        g_wait(sets[p], 1, gbufs[1])
        compute_scatter(sets[p], 1, gbufs[1])
        if do_refill:
            set_issue(sets[p], 2 * i + 4)
        if do_next:
            set_wait(sets[q])
            g_issue(sets[q], 0, gbufs[0])

    set_issue(sets[0], 0)
    set_wait(sets[0])
    g_issue(sets[0], 0, gbufs[0])
    set_issue(sets[1], 2)

    @pl.loop(0, NIT - 2, step=2)
    def _pipe(t):
        pipe_iter(t, 0, 1, True, True)
        pipe_iter(t + 1, 1, 0, True, True)

    pipe_iter(NIT - 2, 0, 1, False, True)
    pipe_iter(NIT - 1, 1, 0, False, False)

    tb = tile_base + NWINF * W
    ct1 = pltpu.async_copy(si_hbm.at[pl.ds(tb, TAIL)], st_t, sgi0)
    ct2 = pltpu.async_copy(di_hbm.at[pl.ds(tb, TAIL)], dt_t, sgi0)
    ct3 = pltpu.async_copy(ef_hbm.at[pl.ds(tb, TAIL)],
                           et_t.at[pl.ds(0, TAIL)], sgi0)
    ct1.wait()
    ct2.wait()
    ct3.wait()
    cl = pltpu.async_copy(l_hbm.at[st_t], l_v0.at[pl.ds(0, TAIL)], sg0)
    cr = pltpu.async_copy(r_hbm.at[dt_t], r_v0.at[pl.ds(0, TAIL)], sg0)
    cl.wait()
    cr.wait()
    dvt = dt_t[pl.ds(0, 16)]
    plsc.addupdate_scatter(deg_v, [dvt], one16)

    @pl.loop(0, TAIL)
    def _tedge(i):
        e = et_t[pl.ds(i, 16)][0]
        for jj in range(D // 16):
            sl = pl.ds(jj * 16, 16)
            l_v0[i, sl] = jnp.maximum(
                l_v0[i, sl] + r_v0[i, sl] + e * wev_v[sl], 0.0)

    pltpu.sync_copy(l_v0.at[pl.ds(0, TAIL)], acc.at[dt_t], add=True)

    plsc.subcore_barrier()
    pltpu.sync_copy(acc.at[pl.ds(s * RPS, RPS)],
                    out_hbm.at[c, pl.ds(s * RPS, RPS)])
    pltpu.sync_copy(deg_v, deg_hbm.at[pl.ds(wid * NPAD, NPAD)])


def _sc_call(l_arr, r_arr, wev, si, di, ef, z):
    mesh = plsc.VectorSubcoreMesh(core_axis_name="c", subcore_axis_name="s")
    cp = pltpu.CompilerParams()
    if "needs_layout_passes" in pltpu.CompilerParams.__dataclass_fields__:
        cp = dataclasses.replace(cp, needs_layout_passes=False)
    run = pl.kernel(
        _sc_body,
        out_type=(
            jax.ShapeDtypeStruct((NC, NPAD, D), jnp.float32),
            jax.ShapeDtypeStruct((NW * NPAD,), jnp.float32),
        ),
        mesh=mesh,
        scratch_types=[
            pltpu.VMEM((2 * W,), jnp.int32),
            pltpu.VMEM((W,), jnp.int32),
            pltpu.VMEM((W,), jnp.int32),
            pltpu.VMEM((2 * W + 16,), jnp.float32),
            pltpu.VMEM((2 * W,), jnp.int32),
            pltpu.VMEM((W,), jnp.int32),
            pltpu.VMEM((W,), jnp.int32),
            pltpu.VMEM((2 * W + 16,), jnp.float32),
            pltpu.VMEM((W, D), jnp.float32),
            pltpu.VMEM((W, D), jnp.float32),
            pltpu.VMEM((W, D), jnp.float32),
            pltpu.VMEM((W, D), jnp.float32),
            pltpu.VMEM((TAIL,), jnp.int32),
            pltpu.VMEM((TAIL,), jnp.int32),
            pltpu.VMEM((TAIL + 16,), jnp.float32),
            pltpu.VMEM((D,), jnp.float32),
            pltpu.VMEM((NPAD,), jnp.float32),
            pltpu.VMEM_SHARED((NPAD, D), jnp.float32),
            pltpu.SemaphoreType.DMA,
            pltpu.SemaphoreType.DMA,
            pltpu.SemaphoreType.DMA,
            pltpu.SemaphoreType.DMA,
        ],
        compiler_params=cp,
    )
    return run(l_arr, r_arr, wev, si, di, ef, z)


def _post_body(acc_ref, degs_ref, right_ref, wft_ref, bf_ref, spost_ref,
               w1a_ref, w1b_ref, b1_ref, w2t_ref, b2_ref, out_ref):
    agg = acc_ref[0] + acc_ref[1]
    conv = jnp.dot(agg, wft_ref[...], preferred_element_type=jnp.float32,
                   precision=_PREC)
    b32 = jnp.broadcast_to(bf_ref[...], (NW, D))
    degb = lax.dot_general(degs_ref[...], b32, (((0,), (0,)), ((), ())),
                           preferred_element_type=jnp.float32,
                           precision=_PREC)
    conv = (conv + degb) * spost_ref[0, 0]
    h = jnp.dot(conv, w1a_ref[...], preferred_element_type=jnp.float32,
                precision=_PREC)
    h = h + jnp.dot(right_ref[...], w1b_ref[...],
                    preferred_element_type=jnp.float32, precision=_PREC)
    h = jnp.maximum(h + b1_ref[...], 0.0)
    out_ref[...] = jnp.dot(h, w2t_ref[...], preferred_element_type=jnp.float32,
                           precision=_PREC) + b2_ref[...]


_PB = 1024


def _post_call(acc, degs, right, wft, bf, spost, w1a, w1b, b1, w2t, b2):
    full = lambda shape: pl.BlockSpec(shape, lambda i: (0,) * len(shape))
    return pl.pallas_call(
        _post_body,
        grid=(NPAD // _PB,),
        in_specs=[
            pl.BlockSpec((NC, _PB, D), lambda i: (0, i, 0)),
            pl.BlockSpec((NW, _PB), lambda i: (0, i)),
            pl.BlockSpec((_PB, D), lambda i: (i, 0)),
            full((D, D)),
            full((1, D)),
            full((1, 1)),
            full((D, D)),
            full((D, D)),
            full((1, D)),
            full((D, D)),
            full((1, D)),
        ],
        out_specs=pl.BlockSpec((_PB, D), lambda i: (i, 0)),
        out_shape=jax.ShapeDtypeStruct((N, D), jnp.float32),
    )(acc, degs, right, wft, bf, spost, w1a, w1b, b1, w2t, b2)


def kernel(left_features, edge_indices, edge_features, right_features,
           scatter_out_size, W_left, b_left, W_edge, W_right, scale_pre,
           W_final, b_final, scale_post, W_out1, b_out1, W_out2, b_out2):
    del scatter_out_size

    l_arr, r_arr, wev = _pre_call(
        left_features, right_features,
        W_left.T, b_left.reshape(1, D), W_right.T,
        W_edge.T, scale_pre.reshape(1, 1))

    zeros = jnp.zeros((RPS, D), dtype=jnp.float32)
    acc, degs = _sc_call(l_arr, r_arr, wev.reshape(D),
                         edge_indices[0], edge_indices[1],
                         edge_features.reshape(E), zeros)

    out = _post_call(
        acc, degs.reshape(NW, NPAD), right_features,
        W_final.T, b_final.reshape(1, D), scale_post.reshape(1, 1),
        W_out1[:, :D].T, W_out1[:, D:].T, b_out1.reshape(1, D),
        W_out2.T, b_out2.reshape(1, D))
    return out


# speedup vs baseline: 1.1963x; 1.1733x over previous
import dataclasses

import jax
import jax.numpy as jnp
from jax import lax
from jax.experimental import pallas as pl
from jax.experimental.pallas import tpu as pltpu
from jax.experimental.pallas import tpu_sc as plsc

N = 10000
D = 128
E = 320000
NC = 2
NS = 16
NW = NC * NS
EPT = E // NW
W = 64
NPAD = 10240
RPS = NPAD // NS

_PREC = jax.lax.Precision.HIGHEST


def _pre_body(left_ref, right_ref, wlt_ref, bl_ref, wrt_ref, wev_ref, sp_ref,
              l_out, r_out, wev_out):
    s = sp_ref[0, 0]
    l_out[...] = (jnp.dot(left_ref[...], wlt_ref[...],
                          preferred_element_type=jnp.float32,
                          precision=_PREC) + bl_ref[...]) * s
    r_out[...] = jnp.dot(right_ref[...], wrt_ref[...],
                         preferred_element_type=jnp.float32,
                         precision=_PREC) * s
    wev_out[...] = wev_ref[...] * s


_QB = 2000


def _pre_call(left, right, wlt, bl, wrt, wev, sp):
    full = lambda shape: pl.BlockSpec(shape, lambda i: (0,) * len(shape))
    return pl.pallas_call(
        _pre_body,
        grid=(N // _QB,),
        in_specs=[
            pl.BlockSpec((_QB, D), lambda i: (i, 0)),
            pl.BlockSpec((_QB, D), lambda i: (i, 0)),
            full((D, D)),
            full((1, D)),
            full((D, D)),
            full((1, D)),
            full((1, 1)),
        ],
        out_specs=(
            pl.BlockSpec((_QB, D), lambda i: (i, 0)),
            pl.BlockSpec((_QB, D), lambda i: (i, 0)),
            full((1, D)),
        ),
        out_shape=(
            jax.ShapeDtypeStruct((N, D), jnp.float32),
            jax.ShapeDtypeStruct((N, D), jnp.float32),
            jax.ShapeDtypeStruct((1, D), jnp.float32),
        ),
    )(left, right, wlt, bl, wrt, wev, sp)


NWINF = 156
TAIL = EPT - NWINF * W
NIT = NWINF // 2


def _sc_body(l_hbm, r_hbm, wev_hbm, si_hbm, di_hbm, ef_hbm, z_hbm,
             out_hbm, deg_hbm,
             si2_0, da_0, db_0, ef2_0,
             si2_1, da_1, db_1, ef2_1,
             l_v0, r_v0, l_v1, r_v1,
             st_t, dt_t, et_t, wev_v,
             deg_v, acc, sgi0, sgi1, sg0, sg1, scat_sem):
    c = lax.axis_index("c")
    s = lax.axis_index("s")
    wid = c * NS + s
    tile_base = wid * EPT

    pltpu.sync_copy(z_hbm, acc.at[pl.ds(s * RPS, RPS)])
    pltpu.sync_copy(wev_hbm, wev_v)

    zero16 = jnp.zeros((16,), jnp.float32)

    @pl.loop(0, NPAD, step=16)
    def _zero_deg(i):
        deg_v[pl.ds(i, 16)] = zero16

    plsc.subcore_barrier()

    one16 = jnp.ones((16,), jnp.float32)
    sets = ((si2_0, da_0, db_0, ef2_0, sgi0),
            (si2_1, da_1, db_1, ef2_1, sgi1))
    gbufs = ((l_v0, r_v0, sg0), (l_v1, r_v1, sg1))

    def set_issue(st, w0):
        si2, da, db, ef2, sem = st
        base = tile_base + w0 * W
        pltpu.async_copy(si_hbm.at[pl.ds(base, 2 * W)], si2, sem)
        pltpu.async_copy(di_hbm.at[pl.ds(base, W)], da, sem)
        pltpu.async_copy(di_hbm.at[pl.ds(base + W, W)], db, sem)
        pltpu.async_copy(ef_hbm.at[pl.ds(base, 2 * W)],
                         ef2.at[pl.ds(0, 2 * W)], sem)

    def set_wait(st):
        si2, da, db, ef2, sem = st
        base = tile_base
        pltpu.make_async_copy(si_hbm.at[pl.ds(base, 2 * W)], si2, sem).wait()
        pltpu.make_async_copy(di_hbm.at[pl.ds(base, W)], da, sem).wait()
        pltpu.make_async_copy(di_hbm.at[pl.ds(base, W)], db, sem).wait()
        pltpu.make_async_copy(ef_hbm.at[pl.ds(base, 2 * W)],
                              ef2.at[pl.ds(0, 2 * W)], sem).wait()

    def g_issue(st, half, gb):
        si2 = st[0]
        l_v, r_v, sem = gb
        didx = st[1] if half == 0 else st[2]
        pltpu.async_copy(l_hbm.at[si2.at[pl.ds(half * W, W)]], l_v, sem)
        pltpu.async_copy(r_hbm.at[didx], r_v, sem)

    def g_wait(st, half, gb):
        si2 = st[0]
        l_v, r_v, sem = gb
        didx = st[1] if half == 0 else st[2]
        pltpu.make_async_copy(l_hbm.at[si2.at[pl.ds(half * W, W)]],
                              l_v, sem).wait()
        pltpu.make_async_copy(r_hbm.at[didx], r_v, sem).wait()

    scsem = scat_sem

    def compute_scatter(st, half, gb):
        _, da, db, ef2, _ = st
        l_v, r_v, _ = gb
        didx = da if half == 0 else db
        eoff = half * W

        @pl.loop(0, W, step=16)
        def _hist(i0):
            dv = didx[pl.ds(i0, 16)]
            plsc.addupdate_scatter(deg_v, [dv], one16)

        wchunks = tuple(wev_v[pl.ds(jj * 16, 16)] for jj in range(D // 16))

        @plsc.parallel_loop(0, W, unroll=4, carry=wchunks)
        def _edge(i, wc):
            e = ef2[pl.ds(eoff + i, 16)][0]
            for jj in range(D // 16):
                sl = pl.ds(jj * 16, 16)
                l_v[i, sl] = jnp.maximum(
                    l_v[i, sl] + r_v[i, sl] + e * wc[jj], 0.0)
            return wc

        pltpu.async_copy(l_v, acc.at[didx], add=True, sem=scsem)

    def scat_wait(st, half, gb):
        didx = st[1] if half == 0 else st[2]
        l_v = gb[0]
        pltpu.make_async_copy(l_v, acc.at[didx], scsem).wait()

    def pipe_iter(i, p, q, do_refill, do_next):
        g_issue(sets[p], 1, gbufs[1])
        g_wait(sets[p], 0, gbufs[0])
        compute_scatter(sets[p], 0, gbufs[0])
        g_wait(sets[p], 1, gbufs[1])
        compute_scatter(sets[p], 1, gbufs[1])
        if do_refill:
            set_issue(sets[p], 2 * i + 4)
        scat_wait(sets[p], 0, gbufs[0])
        if do_next:
            set_wait(sets[q])
            g_issue(sets[q], 0, gbufs[0])
        scat_wait(sets[p], 1, gbufs[1])

    set_issue(sets[0], 0)
    set_wait(sets[0])
    g_issue(sets[0], 0, gbufs[0])
    set_issue(sets[1], 2)

    @pl.loop(0, NIT - 2, step=2)
    def _pipe(t):
        pipe_iter(t, 0, 1, True, True)
        pipe_iter(t + 1, 1, 0, True, True)

    pipe_iter(NIT - 2, 0, 1, False, True)
    pipe_iter(NIT - 1, 1, 0, False, False)

    tb = tile_base + NWINF * W
    ct1 = pltpu.async_copy(si_hbm.at[pl.ds(tb, TAIL)], st_t, sgi0)
    ct2 = pltpu.async_copy(di_hbm.at[pl.ds(tb, TAIL)], dt_t, sgi0)
    ct3 = pltpu.async_copy(ef_hbm.at[pl.ds(tb, TAIL)],
                           et_t.at[pl.ds(0, TAIL)], sgi0)
    ct1.wait()
    ct2.wait()
    ct3.wait()
    cl = pltpu.async_copy(l_hbm.at[st_t], l_v0.at[pl.ds(0, TAIL)], sg0)
    cr = pltpu.async_copy(r_hbm.at[dt_t], r_v0.at[pl.ds(0, TAIL)], sg0)
    cl.wait()
    cr.wait()
    dvt = dt_t[pl.ds(0, 16)]
    plsc.addupdate_scatter(deg_v, [dvt], one16)

    @pl.loop(0, TAIL)
    def _tedge(i):
        e = et_t[pl.ds(i, 16)][0]
        for jj in range(D // 16):
            sl = pl.ds(jj * 16, 16)
            l_v0[i, sl] = jnp.maximum(
                l_v0[i, sl] + r_v0[i, sl] + e * wev_v[sl], 0.0)

    pltpu.sync_copy(l_v0.at[pl.ds(0, TAIL)], acc.at[dt_t], add=True)

    plsc.subcore_barrier()
    pltpu.sync_copy(acc.at[pl.ds(s * RPS, RPS)],
                    out_hbm.at[c, pl.ds(s * RPS, RPS)])
    pltpu.sync_copy(deg_v, deg_hbm.at[pl.ds(wid * NPAD, NPAD)])


def _sc_call(l_arr, r_arr, wev, si, di, ef, z):
    mesh = plsc.VectorSubcoreMesh(core_axis_name="c", subcore_axis_name="s")
    cp = pltpu.CompilerParams()
    if "needs_layout_passes" in pltpu.CompilerParams.__dataclass_fields__:
        cp = dataclasses.replace(cp, needs_layout_passes=False)
    run = pl.kernel(
        _sc_body,
        out_type=(
            jax.ShapeDtypeStruct((NC, NPAD, D), jnp.float32),
            jax.ShapeDtypeStruct((NW * NPAD,), jnp.float32),
        ),
        mesh=mesh,
        scratch_types=[
            pltpu.VMEM((2 * W,), jnp.int32),
            pltpu.VMEM((W,), jnp.int32),
            pltpu.VMEM((W,), jnp.int32),
            pltpu.VMEM((2 * W + 16,), jnp.float32),
            pltpu.VMEM((2 * W,), jnp.int32),
            pltpu.VMEM((W,), jnp.int32),
            pltpu.VMEM((W,), jnp.int32),
            pltpu.VMEM((2 * W + 16,), jnp.float32),
            pltpu.VMEM((W, D), jnp.float32),
            pltpu.VMEM((W, D), jnp.float32),
            pltpu.VMEM((W, D), jnp.float32),
            pltpu.VMEM((W, D), jnp.float32),
            pltpu.VMEM((TAIL,), jnp.int32),
            pltpu.VMEM((TAIL,), jnp.int32),
            pltpu.VMEM((TAIL + 16,), jnp.float32),
            pltpu.VMEM((D,), jnp.float32),
            pltpu.VMEM((NPAD,), jnp.float32),
            pltpu.VMEM_SHARED((NPAD, D), jnp.float32),
            pltpu.SemaphoreType.DMA,
            pltpu.SemaphoreType.DMA,
            pltpu.SemaphoreType.DMA,
            pltpu.SemaphoreType.DMA,
            pltpu.SemaphoreType.DMA,
        ],
        compiler_params=cp,
    )
    return run(l_arr, r_arr, wev, si, di, ef, z)


def _post_body(acc_ref, degs_ref, right_ref, wft_ref, bf_ref, spost_ref,
               w1a_ref, w1b_ref, b1_ref, w2t_ref, b2_ref, out_ref):
    agg = acc_ref[0] + acc_ref[1]
    conv = jnp.dot(agg, wft_ref[...], preferred_element_type=jnp.float32,
                   precision=_PREC)
    b32 = jnp.broadcast_to(bf_ref[...], (NW, D))
    degb = lax.dot_general(degs_ref[...], b32, (((0,), (0,)), ((), ())),
                           preferred_element_type=jnp.float32,
                           precision=_PREC)
    conv = (conv + degb) * spost_ref[0, 0]
    h = jnp.dot(conv, w1a_ref[...], preferred_element_type=jnp.float32,
                precision=_PREC)
    h = h + jnp.dot(right_ref[...], w1b_ref[...],
                    preferred_element_type=jnp.float32, precision=_PREC)
    h = jnp.maximum(h + b1_ref[...], 0.0)
    out_ref[...] = jnp.dot(h, w2t_ref[...], preferred_element_type=jnp.float32,
                           precision=_PREC) + b2_ref[...]


_PB = 1024


def _post_call(acc, degs, right, wft, bf, spost, w1a, w1b, b1, w2t, b2):
    full = lambda shape: pl.BlockSpec(shape, lambda i: (0,) * len(shape))
    return pl.pallas_call(
        _post_body,
        grid=(NPAD // _PB,),
        in_specs=[
            pl.BlockSpec((NC, _PB, D), lambda i: (0, i, 0)),
            pl.BlockSpec((NW, _PB), lambda i: (0, i)),
            pl.BlockSpec((_PB, D), lambda i: (i, 0)),
            full((D, D)),
            full((1, D)),
            full((1, 1)),
            full((D, D)),
            full((D, D)),
            full((1, D)),
            full((D, D)),
            full((1, D)),
        ],
        out_specs=pl.BlockSpec((_PB, D), lambda i: (i, 0)),
        out_shape=jax.ShapeDtypeStruct((N, D), jnp.float32),
    )(acc, degs, right, wft, bf, spost, w1a, w1b, b1, w2t, b2)


def kernel(left_features, edge_indices, edge_features, right_features,
           scatter_out_size, W_left, b_left, W_edge, W_right, scale_pre,
           W_final, b_final, scale_post, W_out1, b_out1, W_out2, b_out2):
    del scatter_out_size

    l_arr, r_arr, wev = _pre_call(
        left_features, right_features,
        W_left.T, b_left.reshape(1, D), W_right.T,
        W_edge.T, scale_pre.reshape(1, 1))

    zeros = jnp.zeros((RPS, D), dtype=jnp.float32)
    acc, degs = _sc_call(l_arr, r_arr, wev.reshape(D),
                         edge_indices[0], edge_indices[1],
                         edge_features.reshape(E), zeros)

    out = _post_call(
        acc, degs.reshape(NW, NPAD), right_features,
        W_final.T, b_final.reshape(1, D), scale_post.reshape(1, 1),
        W_out1[:, :D].T, W_out1[:, D:].T, b_out1.reshape(1, D),
        W_out2.T, b_out2.reshape(1, D))
    return out
